# Initial kernel scaffold; baseline (speedup 1.0000x reference)
import jax
import jax.numpy as jnp
from jax.experimental import pallas as pl


def kernel(x, disease_id, W_share, b_share, W_private, b_private, W_transfer, b_transfer, W_gate, W_u_share, b_u_share, W_logvar_share, b_logvar_share, W_u_private, b_u_private, W_logvar_private, b_logvar_private, W_u_transfer, b_u_transfer, W_logvar_transfer, b_logvar_transfer):
    raise NotImplementedError("write your pallas kernel here")


# speedup vs baseline: 1.0836x; 1.0836x over previous
import jax
import jax.numpy as jnp
from jax.experimental import pallas as pl
from jax.experimental.pallas import tpu as pltpu

N_DIS = 4
N_SHARE = 2
N_PRIV = 2
N_TRANS = 16
N_GROUP = 4
GSIZE = N_TRANS // N_GROUP
D = 768
N = 2048
BN = 256
NB = N // BN
NUNITS = N_SHARE + N_DIS * N_PRIV + N_TRANS


def _moe_body(x_ref, dis_ref, wg_ref, wstack_ref, bstack_ref, whead_ref,
              bhead_ref, u_ref, lv_ref, fs_acc, fp_acc, ft_acc, cw_ref, xb_ref):
    k = pl.program_id(1)

    @pl.when(k == 0)
    def _init():
        x32 = x_ref[...]
        xb_ref[...] = x32.astype(jnp.bfloat16)
        fs_acc[...] = jnp.zeros((BN, D), jnp.float32)
        fp_acc[...] = jnp.zeros((BN, D), jnp.float32)
        ft_acc[...] = jnp.zeros((BN, D), jnp.float32)
        logits = jax.lax.dot_general(
            x32, wg_ref[...], (((1,), (0,)), ((), ())),
            precision=jax.lax.Precision.HIGHEST,
            preferred_element_type=jnp.float32)
        m = jnp.max(logits, axis=-1, keepdims=True)
        p = jnp.exp(logits - m)
        scores = p / jnp.sum(p, axis=-1, keepdims=True)
        it16 = jax.lax.broadcasted_iota(jnp.int32, (BN, N_TRANS), 1)
        gid16 = it16 // GSIZE
        neg = jnp.float32(-jnp.inf)
        gm = [jnp.max(jnp.where(gid16 == g, scores, neg), axis=-1, keepdims=True)
              for g in range(N_GROUP)]
        bv = jnp.full((BN, 1), neg, jnp.float32)
        bg = jnp.zeros((BN, 1), jnp.int32)
        for g in range(N_GROUP - 1, -1, -1):
            c = gm[g] >= bv
            bv = jnp.where(c, gm[g], bv)
            bg = jnp.where(c, g, bg)
        g1 = bg
        bv2 = jnp.full((BN, 1), neg, jnp.float32)
        bg2 = jnp.zeros((BN, 1), jnp.int32)
        for g in range(N_GROUP - 1, -1, -1):
            vg = jnp.where(g1 == g, neg, gm[g])
            c = vg >= bv2
            bv2 = jnp.where(c, vg, bv2)
            bg2 = jnp.where(c, g, bg2)
        g2 = bg2
        gmask = (gid16 == g1) | (gid16 == g2)
        masked = jnp.where(gmask, scores, 0.0)
        m1 = jnp.max(masked, axis=-1, keepdims=True)
        am1 = jnp.min(jnp.where(masked == m1, it16, N_TRANS), axis=-1,
                      keepdims=True)
        masked2 = jnp.where(it16 == am1, neg, masked)
        m2 = jnp.max(masked2, axis=-1, keepdims=True)
        am2 = jnp.min(jnp.where(masked2 == m2, it16, N_TRANS), axis=-1,
                      keepdims=True)
        cw_ref[...] = (jnp.where(it16 == am1, m1, 0.0)
                       + jnp.where(it16 == am2, m2, 0.0))

    y = jnp.dot(xb_ref[...], wstack_ref[0], preferred_element_type=jnp.float32)
    y = jnp.maximum(y + bstack_ref[0], 0.0)

    @pl.when(k < N_SHARE)
    def _share():
        fs_acc[...] += y

    @pl.when((k >= N_SHARE) & (k < N_SHARE + N_DIS * N_PRIV))
    def _priv():
        d = (k - N_SHARE) // N_PRIV
        mask = dis_ref[:, :1] == d
        fp_acc[...] += jnp.where(mask, y, 0.0)

    @pl.when(k >= N_SHARE + N_DIS * N_PRIV)
    def _trans():
        e = k - (N_SHARE + N_DIS * N_PRIV)
        it16 = jax.lax.broadcasted_iota(jnp.int32, (BN, N_TRANS), 1)
        wcol = jnp.sum(jnp.where(it16 == e, cw_ref[...], 0.0), axis=-1,
                       keepdims=True)
        ft_acc[...] += wcol * y

    @pl.when(k == NUNITS - 1)
    def _heads():
        fsb = fs_acc[...].astype(jnp.bfloat16)
        fpb = fp_acc[...].astype(jnp.bfloat16)
        ftb = ft_acc[...].astype(jnp.bfloat16)
        u = (jnp.dot(fsb, whead_ref[0], preferred_element_type=jnp.float32)
             + jnp.dot(fpb, whead_ref[1], preferred_element_type=jnp.float32)
             + jnp.dot(ftb, whead_ref[2], preferred_element_type=jnp.float32))
        lv = (jnp.dot(fsb, whead_ref[3], preferred_element_type=jnp.float32)
              + jnp.dot(fpb, whead_ref[4], preferred_element_type=jnp.float32)
              + jnp.dot(ftb, whead_ref[5], preferred_element_type=jnp.float32))
        u_ref[...] = u + bhead_ref[0:1] + bhead_ref[1:2] + bhead_ref[2:3]
        lv_ref[...] = lv + bhead_ref[3:4] + bhead_ref[4:5] + bhead_ref[5:6]


def kernel(x, disease_id, W_share, b_share, W_private, b_private, W_transfer,
           b_transfer, W_gate, W_u_share, b_u_share, W_logvar_share,
           b_logvar_share, W_u_private, b_u_private, W_logvar_private,
           b_logvar_private, W_u_transfer, b_u_transfer, W_logvar_transfer,
           b_logvar_transfer):
    wstack = jnp.concatenate(
        [W_share, W_private.reshape(N_DIS * N_PRIV, D, D), W_transfer],
        axis=0).astype(jnp.bfloat16)
    bstack = jnp.concatenate(
        [b_share, b_private.reshape(N_DIS * N_PRIV, D), b_transfer],
        axis=0).reshape(NUNITS, 1, D)
    whead = jnp.stack([W_u_share, W_u_private, W_u_transfer,
                       W_logvar_share, W_logvar_private, W_logvar_transfer],
                      axis=0).astype(jnp.bfloat16)
    bhead = jnp.stack([b_u_share, b_u_private, b_u_transfer,
                       b_logvar_share, b_logvar_private, b_logvar_transfer],
                      axis=0)
    dis2d = jnp.broadcast_to(disease_id[:, None], (N, 128))

    u, lv = pl.pallas_call(
        _moe_body,
        grid=(NB, NUNITS),
        in_specs=[
            pl.BlockSpec((BN, D), lambda i, k: (i, 0)),
            pl.BlockSpec((BN, 128), lambda i, k: (i, 0)),
            pl.BlockSpec((D, N_TRANS), lambda i, k: (0, 0)),
            pl.BlockSpec((1, D, D), lambda i, k: (k, 0, 0)),
            pl.BlockSpec((1, 1, D), lambda i, k: (k, 0, 0)),
            pl.BlockSpec((6, D, D), lambda i, k: (0, 0, 0)),
            pl.BlockSpec((6, D), lambda i, k: (0, 0)),
        ],
        out_specs=[
            pl.BlockSpec((BN, D), lambda i, k: (i, 0)),
            pl.BlockSpec((BN, D), lambda i, k: (i, 0)),
        ],
        out_shape=[
            jax.ShapeDtypeStruct((N, D), jnp.float32),
            jax.ShapeDtypeStruct((N, D), jnp.float32),
        ],
        scratch_shapes=[
            pltpu.VMEM((BN, D), jnp.float32),
            pltpu.VMEM((BN, D), jnp.float32),
            pltpu.VMEM((BN, D), jnp.float32),
            pltpu.VMEM((BN, N_TRANS), jnp.float32),
            pltpu.VMEM((BN, D), jnp.bfloat16),
        ],
    )(x, dis2d, W_gate, wstack, bstack, whead, bhead)
    return (u, lv)


# speedup vs baseline: 1.8153x; 1.6752x over previous
import jax
import jax.numpy as jnp
from jax.experimental import pallas as pl
from jax.experimental.pallas import tpu as pltpu

N_DIS = 4
N_SHARE = 2
N_PRIV = 2
N_TRANS = 16
N_GROUP = 4
GSIZE = N_TRANS // N_GROUP
D = 768
N = 2048
NEXP = N_SHARE + N_DIS * N_PRIV + N_TRANS
NSTEPS = NEXP + 6


def _routing(x32, wg):
    logits = jax.lax.dot_general(
        x32, wg, (((1,), (0,)), ((), ())),
        precision=jax.lax.Precision.HIGHEST,
        preferred_element_type=jnp.float32)
    m = jnp.max(logits, axis=-1, keepdims=True)
    p = jnp.exp(logits - m)
    scores = p / jnp.sum(p, axis=-1, keepdims=True)
    n = x32.shape[0]
    it16 = jax.lax.broadcasted_iota(jnp.int32, (n, N_TRANS), 1)
    gid16 = it16 // GSIZE
    neg = jnp.float32(-jnp.inf)
    gm = [jnp.max(jnp.where(gid16 == g, scores, neg), axis=-1, keepdims=True)
          for g in range(N_GROUP)]
    bv = jnp.full((n, 1), neg, jnp.float32)
    bg = jnp.zeros((n, 1), jnp.int32)
    for g in range(N_GROUP - 1, -1, -1):
        c = gm[g] >= bv
        bv = jnp.where(c, gm[g], bv)
        bg = jnp.where(c, g, bg)
    g1 = bg
    bv2 = jnp.full((n, 1), neg, jnp.float32)
    bg2 = jnp.zeros((n, 1), jnp.int32)
    for g in range(N_GROUP - 1, -1, -1):
        vg = jnp.where(g1 == g, neg, gm[g])
        c = vg >= bv2
        bv2 = jnp.where(c, vg, bv2)
        bg2 = jnp.where(c, g, bg2)
    gmask = (gid16 == g1) | (gid16 == bg2)
    masked = jnp.where(gmask, scores, 0.0)
    m1 = jnp.max(masked, axis=-1, keepdims=True)
    am1 = jnp.min(jnp.where(masked == m1, it16, N_TRANS), axis=-1,
                  keepdims=True)
    masked2 = jnp.where(it16 == am1, neg, masked)
    m2 = jnp.max(masked2, axis=-1, keepdims=True)
    am2 = jnp.min(jnp.where(masked2 == m2, it16, N_TRANS), axis=-1,
                  keepdims=True)
    return (jnp.where(it16 == am1, m1, 0.0)
            + jnp.where(it16 == am2, m2, 0.0))


def _moe_body(x_ref, dis_ref, wg_ref, wstack_ref, bstack_ref, bhead_ref,
              u_ref, lv_ref, fs_acc, fp_acc, ft_acc, cw_ref, xb_ref):
    k = pl.program_id(0)

    @pl.when(k == 0)
    def _init():
        x32 = x_ref[...]
        xb_ref[...] = x32.astype(jnp.bfloat16)
        fs_acc[...] = jnp.zeros((N, D), jnp.float32)
        fp_acc[...] = jnp.zeros((N, D), jnp.float32)
        ft_acc[...] = jnp.zeros((N, D), jnp.float32)
        cw_ref[...] = _routing(x32, wg_ref[...])

    @pl.when(k < NEXP)
    def _expert():
        y = jnp.dot(xb_ref[...], wstack_ref[0],
                    preferred_element_type=jnp.float32)
        y = jnp.maximum(y + bstack_ref[0], 0.0)

        @pl.when(k < N_SHARE)
        def _share():
            fs_acc[...] += y

        @pl.when((k >= N_SHARE) & (k < N_SHARE + N_DIS * N_PRIV))
        def _priv():
            d = (k - N_SHARE) // N_PRIV
            mask = dis_ref[:, :1] == d
            fp_acc[...] += jnp.where(mask, y, 0.0)

        @pl.when(k >= N_SHARE + N_DIS * N_PRIV)
        def _trans():
            e = k - (N_SHARE + N_DIS * N_PRIV)
            it16 = jax.lax.broadcasted_iota(jnp.int32, (N, N_TRANS), 1)
            wcol = jnp.sum(jnp.where(it16 == e, cw_ref[...], 0.0), axis=-1,
                           keepdims=True)
            ft_acc[...] += wcol * y

    def _head(step, src_acc, out_ref, first, bias):
        @pl.when(k == step)
        def _():
            h = jnp.dot(src_acc[...].astype(jnp.bfloat16), wstack_ref[0],
                        preferred_element_type=jnp.float32)
            if first:
                out_ref[...] = h + bias
            else:
                out_ref[...] += h

    bias_u = bhead_ref[0:1] + bhead_ref[1:2] + bhead_ref[2:3]
    bias_lv = bhead_ref[3:4] + bhead_ref[4:5] + bhead_ref[5:6]
    _head(NEXP + 0, fs_acc, u_ref, True, bias_u)
    _head(NEXP + 1, fp_acc, u_ref, False, None)
    _head(NEXP + 2, ft_acc, u_ref, False, None)
    _head(NEXP + 3, fs_acc, lv_ref, True, bias_lv)
    _head(NEXP + 4, fp_acc, lv_ref, False, None)
    _head(NEXP + 5, ft_acc, lv_ref, False, None)


def kernel(x, disease_id, W_share, b_share, W_private, b_private, W_transfer,
           b_transfer, W_gate, W_u_share, b_u_share, W_logvar_share,
           b_logvar_share, W_u_private, b_u_private, W_logvar_private,
           b_logvar_private, W_u_transfer, b_u_transfer, W_logvar_transfer,
           b_logvar_transfer):
    wstack = jnp.concatenate(
        [W_share, W_private.reshape(N_DIS * N_PRIV, D, D), W_transfer,
         W_u_share[None], W_u_private[None], W_u_transfer[None],
         W_logvar_share[None], W_logvar_private[None],
         W_logvar_transfer[None]],
        axis=0).astype(jnp.bfloat16)
    bstack = jnp.concatenate(
        [b_share, b_private.reshape(N_DIS * N_PRIV, D), b_transfer],
        axis=0).reshape(NEXP, 1, D)
    bhead = jnp.stack([b_u_share, b_u_private, b_u_transfer,
                       b_logvar_share, b_logvar_private, b_logvar_transfer],
                      axis=0)
    dis2d = jnp.broadcast_to(disease_id[:, None], (N, 128))

    u, lv = pl.pallas_call(
        _moe_body,
        grid=(NSTEPS,),
        in_specs=[
            pl.BlockSpec((N, D), lambda k: (0, 0)),
            pl.BlockSpec((N, 128), lambda k: (0, 0)),
            pl.BlockSpec((D, N_TRANS), lambda k: (0, 0)),
            pl.BlockSpec((1, D, D), lambda k: (k, 0, 0)),
            pl.BlockSpec((1, 1, D), lambda k: (jnp.minimum(k, NEXP - 1), 0, 0)),
            pl.BlockSpec((6, D), lambda k: (0, 0)),
        ],
        out_specs=[
            pl.BlockSpec((N, D), lambda k: (0, 0)),
            pl.BlockSpec((N, D), lambda k: (0, 0)),
        ],
        out_shape=[
            jax.ShapeDtypeStruct((N, D), jnp.float32),
            jax.ShapeDtypeStruct((N, D), jnp.float32),
        ],
        scratch_shapes=[
            pltpu.VMEM((N, D), jnp.float32),
            pltpu.VMEM((N, D), jnp.float32),
            pltpu.VMEM((N, D), jnp.float32),
            pltpu.VMEM((N, N_TRANS), jnp.float32),
            pltpu.VMEM((N, D), jnp.bfloat16),
        ],
    )(x, dis2d, W_gate, wstack, bstack, bhead)
    return (u, lv)
